# baseline (speedup 1.0000x reference)
import functools

import jax
import jax.numpy as jnp
from jax import lax
from jax.experimental import pallas as pl
from jax.experimental.pallas import tpu as pltpu
from jax.experimental.pallas import tpu_sc as plsc

N_FIELDS = 26
VOCAB = 100000
EMBED_DIM = 64
BATCH = 4096

_NC = 2
_NS = 16
_NW = _NC * _NS
_ROWS = BATCH * N_FIELDS
_BPW = BATCH // _NW
_RPW = _BPW * N_FIELDS
_CB = 4
_CROWS = _CB * N_FIELDS
_NCHUNK = _BPW // _CB
_LANES = 16


@functools.partial(
    pl.kernel,
    out_type=jax.ShapeDtypeStruct((_ROWS, EMBED_DIM), jnp.float32),
    mesh=plsc.VectorSubcoreMesh(core_axis_name="c", subcore_axis_name="s"),
    scratch_types=[
        pltpu.VMEM((_BPW, N_FIELDS), jnp.int32),
        pltpu.VMEM((_CROWS, EMBED_DIM), jnp.float32),
        pltpu.VMEM((_CROWS, EMBED_DIM), jnp.float32),
        pltpu.SemaphoreType.DMA,
        pltpu.SemaphoreType.DMA,
        pltpu.SemaphoreType.DMA,
        pltpu.SemaphoreType.DMA,
    ],
    compiler_params=pltpu.CompilerParams(use_tc_tiling_on_sc=True),
)
def _gather(x_hbm, tab_hbm, out_hbm, xbuf, r0, r1, g0, g1, o0, o1):
    wid = lax.axis_index("s") * _NC + lax.axis_index("c")
    bbase = wid * _BPW
    rbase = wid * _RPW
    rbufs = (r0, r1)
    gsems = (g0, g1)
    osems = (o0, o1)

    pltpu.sync_copy(x_hbm.at[pl.ds(bbase, _BPW)], xbuf)

    def fire(m, b):
        for bb in range(_CB):
            row = m * _CB + bb
            v0 = xbuf[row, pl.ds(0, _LANES)]
            v1 = xbuf[row, pl.ds(N_FIELDS - _LANES, _LANES)]
            for f in range(N_FIELDS):
                v = v0[f] if f < _LANES else v1[f - (N_FIELDS - _LANES)]
                pltpu.make_async_copy(
                    tab_hbm.at[f, pl.ds(v, 1)],
                    rbufs[b].at[pl.ds(bb * N_FIELDS + f, 1)],
                    gsems[b]).start()

    def gwait(b):
        pltpu.make_async_copy(
            out_hbm.at[pl.ds(0, _CROWS)], rbufs[b], gsems[b]).wait()

    def ostart(m, b):
        pltpu.make_async_copy(
            rbufs[b], out_hbm.at[pl.ds(rbase + m * _CROWS, _CROWS)],
            osems[b]).start()

    def owait(b):
        pltpu.make_async_copy(
            rbufs[b], out_hbm.at[pl.ds(rbase, _CROWS)], osems[b]).wait()

    fire(0, 0)
    fire(1, 1)

    def pipe_body(i, carry):
        for b in range(2):
            m = 2 * i + b
            gwait(b)
            ostart(m, b)
            owait(b)
            fire(m + 2, b)
        return carry

    lax.fori_loop(0, _NCHUNK // 2 - 1, pipe_body, 0)

    for m in (_NCHUNK - 2, _NCHUNK - 1):
        b = m % 2
        gwait(b)
        ostart(m, b)
        owait(b)


def kernel(x, tables):
    out = _gather(x.astype(jnp.int32), tables)
    return out.reshape(BATCH, N_FIELDS * EMBED_DIM)


# speedup vs baseline: 1.6959x; 1.6959x over previous
import functools

import jax
import jax.numpy as jnp
from jax import lax
from jax.experimental import pallas as pl
from jax.experimental.pallas import tpu as pltpu
from jax.experimental.pallas import tpu_sc as plsc

N_FIELDS = 26
VOCAB = 100000
EMBED_DIM = 64
BATCH = 4096

_NC = 2
_NS = 16
_NW = _NC * _NS
_ROWS = BATCH * N_FIELDS
_BPW = BATCH // _NW
_RPW = _BPW * N_FIELDS
_CB = 4
_CROWS = _CB * N_FIELDS
_NCHUNK = _BPW // _CB
_LANES = 16


@functools.partial(
    pl.kernel,
    out_type=jax.ShapeDtypeStruct((_ROWS, EMBED_DIM), jnp.float32),
    mesh=plsc.VectorSubcoreMesh(core_axis_name="c", subcore_axis_name="s"),
    scratch_types=[
        pltpu.VMEM((_BPW, N_FIELDS), jnp.int32),
        pltpu.VMEM((_CROWS, EMBED_DIM), jnp.float32),
        pltpu.VMEM((_CROWS, EMBED_DIM), jnp.float32),
        pltpu.SemaphoreType.DMA,
        pltpu.SemaphoreType.DMA,
        pltpu.SemaphoreType.DMA,
        pltpu.SemaphoreType.DMA,
    ],
    compiler_params=pltpu.CompilerParams(use_tc_tiling_on_sc=True),
)
def _gather(x_hbm, tab_hbm, out_hbm, xbuf, r0, r1, g0, g1, o0, o1):
    wid = lax.axis_index("s") * _NC + lax.axis_index("c")
    bbase = wid * _BPW
    rbase = wid * _RPW
    rbufs = (r0, r1)
    gsems = (g0, g1)
    osems = (o0, o1)

    pltpu.sync_copy(x_hbm.at[pl.ds(bbase, _BPW)], xbuf)

    def fire(m, b):
        for bb in range(_CB):
            row = m * _CB + bb
            v0 = xbuf[row, pl.ds(0, _LANES)]
            v1 = xbuf[row, pl.ds(N_FIELDS - _LANES, _LANES)]
            for f in range(N_FIELDS):
                v = v0[f] if f < _LANES else v1[f - (N_FIELDS - _LANES)]
                t = lax.shift_right_logical(v, 3) + f * (VOCAB // 8)
                s = lax.bitwise_and(v, 7)
                pltpu.make_async_copy(
                    tab_hbm.at[t, pl.ds(s, 1)],
                    rbufs[b].at[pl.ds(bb * N_FIELDS + f, 1)],
                    gsems[b]).start()

    def gwait(b):
        pltpu.make_async_copy(
            out_hbm.at[pl.ds(0, _CROWS)], rbufs[b], gsems[b]).wait()

    def ostart(m, b):
        pltpu.make_async_copy(
            rbufs[b], out_hbm.at[pl.ds(rbase + m * _CROWS, _CROWS)],
            osems[b]).start()

    def owait(b):
        pltpu.make_async_copy(
            rbufs[b], out_hbm.at[pl.ds(rbase, _CROWS)], osems[b]).wait()

    fire(0, 0)
    fire(1, 1)

    def pipe_body(i, carry):
        for b in range(2):
            m = 2 * i + b
            gwait(b)
            ostart(m, b)
            owait(b)
            fire(m + 2, b)
        return carry

    lax.fori_loop(0, _NCHUNK // 2 - 1, pipe_body, 0)

    for m in (_NCHUNK - 2, _NCHUNK - 1):
        b = m % 2
        gwait(b)
        ostart(m, b)
        owait(b)


def kernel(x, tables):
    tab = tables.reshape(N_FIELDS * VOCAB // 8, 8, EMBED_DIM)
    out = _gather(x.astype(jnp.int32), tab)
    return out.reshape(BATCH, N_FIELDS * EMBED_DIM)


# speedup vs baseline: 1.7460x; 1.0295x over previous
import functools

import jax
import jax.numpy as jnp
from jax import lax
from jax.experimental import pallas as pl
from jax.experimental.pallas import tpu as pltpu
from jax.experimental.pallas import tpu_sc as plsc

N_FIELDS = 26
VOCAB = 100000
EMBED_DIM = 64
BATCH = 4096
OUT_D = N_FIELDS * EMBED_DIM

_NC = 2
_NS = 16
_NW = _NC * _NS
_BPW = BATCH // _NW
_CB = 8
_CROWS = _CB * N_FIELDS
_NCHUNK = _BPW // _CB
_LANES = 16
_VEC = EMBED_DIM // _LANES


@functools.partial(
    pl.kernel,
    out_type=jax.ShapeDtypeStruct((BATCH, OUT_D), jnp.float32),
    mesh=plsc.VectorSubcoreMesh(core_axis_name="c", subcore_axis_name="s"),
    scratch_types=[
        pltpu.VMEM((_BPW, N_FIELDS), jnp.int32),
        pltpu.VMEM((_CROWS // 8, 8, EMBED_DIM), jnp.float32),
        pltpu.VMEM((_CROWS // 8, 8, EMBED_DIM), jnp.float32),
        pltpu.VMEM((_CB, OUT_D), jnp.float32),
        pltpu.VMEM((_CB, OUT_D), jnp.float32),
        pltpu.SemaphoreType.DMA,
        pltpu.SemaphoreType.DMA,
        pltpu.SemaphoreType.DMA,
        pltpu.SemaphoreType.DMA,
    ],
    compiler_params=pltpu.CompilerParams(use_tc_tiling_on_sc=True),
)
def _gather(x_hbm, tab_hbm, out_hbm, xbuf, r0, r1, ob0, ob1,
            g0, g1, o0, o1):
    wid = lax.axis_index("s") * _NC + lax.axis_index("c")
    bbase = wid * _BPW
    rbufs = (r0, r1)
    obufs = (ob0, ob1)
    gsems = (g0, g1)
    osems = (o0, o1)

    pltpu.sync_copy(x_hbm.at[pl.ds(bbase, _BPW)], xbuf)

    def fire(m, b):
        for bb in range(_CB):
            row = m * _CB + bb
            v0 = xbuf[row, pl.ds(0, _LANES)]
            v1 = xbuf[row, pl.ds(N_FIELDS - _LANES, _LANES)]
            for f in range(N_FIELDS):
                v = v0[f] if f < _LANES else v1[f - (N_FIELDS - _LANES)]
                t = lax.shift_right_logical(v, 3) + f * (VOCAB // 8)
                s = lax.bitwise_and(v, 7)
                r = bb * N_FIELDS + f
                pltpu.make_async_copy(
                    tab_hbm.at[t, pl.ds(s, 1)],
                    rbufs[b].at[r // 8, pl.ds(r % 8, 1)],
                    gsems[b]).start()

    def gwait(b):
        pltpu.make_async_copy(
            tab_hbm.at[pl.ds(0, _CROWS // 8)], rbufs[b], gsems[b]).wait()

    def repack(b):
        for bb in range(_CB):
            for f in range(N_FIELDS):
                r = bb * N_FIELDS + f
                for k in range(_VEC):
                    obufs[b][bb, pl.ds(f * EMBED_DIM + k * _LANES, _LANES)] = (
                        rbufs[b][r // 8, r % 8, pl.ds(k * _LANES, _LANES)])

    def ostart(m, b):
        pltpu.make_async_copy(
            obufs[b], out_hbm.at[pl.ds(bbase + m * _CB, _CB)],
            osems[b]).start()

    def owait(b):
        pltpu.make_async_copy(
            obufs[b], out_hbm.at[pl.ds(bbase, _CB)], osems[b]).wait()

    fire(0, 0)
    fire(1, 1)

    def pipe_body(i, carry):
        for b in range(2):
            m = 2 * i + b
            gwait(b)
            repack(b)
            ostart(m, b)
            owait(b)
            fire(m + 2, b)
        return carry

    lax.fori_loop(0, _NCHUNK // 2 - 1, pipe_body, 0)

    for m in (_NCHUNK - 2, _NCHUNK - 1):
        b = m % 2
        gwait(b)
        repack(b)
        ostart(m, b)
        owait(b)


def kernel(x, tables):
    tab = tables.reshape(N_FIELDS * VOCAB // 8, 8, EMBED_DIM)
    return _gather(x.astype(jnp.int32), tab)
